# baseline (speedup 1.0000x reference)
import functools

import jax
import jax.numpy as jnp
from jax import lax
from jax.experimental import pallas as pl
from jax.experimental.pallas import tpu as pltpu
from jax.experimental.pallas import tpu_sc as plsc

N_POINTS = 16384
NB_NEIGHBORS = 16
N_SAMPLES = N_POINTS // NB_NEIGHBORS
RADIUS = 0.08
R2 = RADIUS * RADIUS
MAXK = 32
CAP = 128
STAGE_W = 160
D_HID = 128
D_OUT = 256

NUM_WORKERS = 32
ANCH_PER_W = N_SAMPLES // NUM_WORKERS


def _fps_kernel(x_ref, y_ref, z_ref, fx_ref, fy_ref, fz_ref):
    x = x_ref[...]
    y = y_ref[...]
    z = z_ref[...]
    row_io = lax.broadcasted_iota(jnp.int32, (128, 128), 0)
    col_io = lax.broadcasted_iota(jnp.int32, (128, 128), 1)
    flat_io = row_io * 128 + col_io

    x0 = x[0, 0]
    y0 = y[0, 0]
    z0 = z[0, 0]
    dx = x - x0
    dy = y - y0
    dz = z - z0
    dists0 = (dx * dx + dy * dy) + dz * dz

    sio = lax.broadcasted_iota(jnp.int32, (8, 128), 0) * 128 + \
        lax.broadcasted_iota(jnp.int32, (8, 128), 1)
    sel0 = sio == 0
    fxv0 = jnp.where(sel0, x0, jnp.float32(0.0))
    fyv0 = jnp.where(sel0, y0, jnp.float32(0.0))
    fzv0 = jnp.where(sel0, z0, jnp.float32(0.0))

    def body(i, st):
        dists, fxv, fyv, fzv = st
        maxv = jnp.max(dists)
        nxt = jnp.min(jnp.where(dists == maxv, flat_io, jnp.int32(N_POINTS)))
        one_hot = flat_io == nxt
        px = jnp.sum(jnp.where(one_hot, x, jnp.float32(0.0)))
        py = jnp.sum(jnp.where(one_hot, y, jnp.float32(0.0)))
        pz = jnp.sum(jnp.where(one_hot, z, jnp.float32(0.0)))
        ddx = x - px
        ddy = y - py
        ddz = z - pz
        d = (ddx * ddx + ddy * ddy) + ddz * ddz
        dists = jnp.minimum(dists, d)
        sel = sio == i
        fxv = jnp.where(sel, px, fxv)
        fyv = jnp.where(sel, py, fyv)
        fzv = jnp.where(sel, pz, fzv)
        return (dists, fxv, fyv, fzv)

    _, fxv, fyv, fzv = lax.fori_loop(
        1, N_SAMPLES, body, (dists0, fxv0, fyv0, fzv0))
    fx_ref[...] = fxv
    fy_ref[...] = fyv
    fz_ref[...] = fzv


def _run_fps(points):
    x2 = points[:, 0].reshape(128, 128)
    y2 = points[:, 1].reshape(128, 128)
    z2 = points[:, 2].reshape(128, 128)
    out = jax.ShapeDtypeStruct((8, 128), jnp.float32)
    fx, fy, fz = pl.pallas_call(
        _fps_kernel,
        out_shape=(out, out, out),
    )(x2, y2, z2)
    return fx.reshape(N_SAMPLES), fy.reshape(N_SAMPLES), fz.reshape(N_SAMPLES)


CBLK = 1024
NJ = N_POINTS // CBLK
GRP = 256
NGRP_BLK = CBLK // GRP


def _make_s12():
    import numpy as np
    s = np.zeros((CBLK, 128), np.float32)
    p = np.arange(CBLK)
    s[p, p // 16] = 2.0 ** (p % 16)
    s[p, 64 + p // GRP] = 1.0
    return jnp.asarray(s, jnp.bfloat16)


def _make_s3():
    import numpy as np
    s = np.zeros((64, 128), np.float32)
    c = np.arange(64)
    s[c, 68 + c // 16] = 2.0 ** (c % 16)
    return jnp.asarray(s, jnp.bfloat16)


def _d2_kernel(fps_ref, ptt_ref, s12_ref, s3_ref, d2_ref, ids_ref):
    f = fps_ref[...]
    ptt = ptt_ref[...]
    q2 = jnp.sum(f * f, axis=1, keepdims=True)
    p2 = jnp.sum(ptt * ptt, axis=0, keepdims=True)
    mm = jnp.dot(f, ptt, preferred_element_type=jnp.float32)
    d2 = jnp.maximum((q2 + p2) - 2.0 * mm, 0.0)
    d2_ref[...] = d2
    mask = (d2 <= jnp.float32(R2)).astype(jnp.bfloat16)
    ids1 = jnp.dot(mask, s12_ref[...], preferred_element_type=jnp.float32)
    hit01 = (ids1[:, 0:64] > 0.0).astype(jnp.bfloat16)
    ids_ref[...] = ids1 + jnp.dot(hit01, s3_ref[...],
                                  preferred_element_type=jnp.float32)


def _run_d2(fps_points, ptt, s12, s3):
    return pl.pallas_call(
        _d2_kernel,
        grid=(8, NJ),
        in_specs=[
            pl.BlockSpec((128, 3), lambda i, j: (i, 0)),
            pl.BlockSpec((3, CBLK), lambda i, j: (0, j)),
            pl.BlockSpec((CBLK, 128), lambda i, j: (0, 0)),
            pl.BlockSpec((64, 128), lambda i, j: (0, 0)),
        ],
        out_specs=[
            pl.BlockSpec((128, CBLK), lambda i, j: (i, j)),
            pl.BlockSpec((128, 128), lambda i, j: (i, j)),
        ],
        out_shape=[
            jax.ShapeDtypeStruct((N_SAMPLES, N_POINTS), jnp.float32),
            jax.ShapeDtypeStruct((N_SAMPLES, NJ * 128), jnp.float32),
        ],
    )(fps_points, ptt, s12, s3)


def _make_perm_tabs():
    import numpy as np
    tl = np.zeros((256, 16), np.int32)
    th = np.zeros((256, 16), np.int32)
    for m in range(256):
        bits = [b for b in range(8) if (m >> b) & 1]
        tl[m, :len(bits)] = bits
        th[m, :len(bits)] = [b + 8 for b in bits]
        th[m, 8:15] = 8
        tl[m, 15] = len(bits)
        th[m, 15] = len(bits)
    return jnp.asarray(tl.reshape(-1)), jnp.asarray(th.reshape(-1))


def _sc_compact_body(d2_hbm, ids_hbm, px_hbm, py_hbm, pz_hbm,
                     tabl_hbm, tabh_hbm,
                     rx_out, ry_out, rz_out, dc_out,
                     px_v, py_v, pz_v, row_v, ids_v, tabl_v, tabh_v,
                     rx_st, ry_st, rz_st, dc_st, ptr_sm):
    wid = lax.axis_index("c") * 16 + lax.axis_index("s")
    base = wid * ANCH_PER_W

    pltpu.sync_copy(px_hbm, px_v)
    pltpu.sync_copy(py_hbm, py_v)
    pltpu.sync_copy(pz_hbm, pz_v)
    pltpu.sync_copy(tabl_hbm, tabl_v)
    pltpu.sync_copy(tabh_hbm, tabh_v)

    pad = jnp.full((16,), 1e30, jnp.float32)
    iota16 = lax.iota(jnp.int32, 16)
    c15 = jnp.full((16,), 15, jnp.int32)

    def per_anchor(i, _):
        a = base + i
        pltpu.sync_copy(d2_hbm.at[a], row_v)
        pltpu.sync_copy(ids_hbm.at[a], ids_v)
        ptr_sm[0] = 0
        for k in range(STAGE_W // 16):
            dc_st[pl.ds(k * 16, 16)] = pad

        def jblock(j, _j):
            gvec = ids_v[pl.ds(j * 128 + 64, 16)]
            for g in range(NGRP_BLK):
                gmf = gvec[4 + g]

                @pl.when(gmf > jnp.float32(0))
                def _(j=j, g=g):
                    gm = gvec[4 + g].astype(jnp.int32)
                    growl = tabl_v[pl.ds((gm % 256) * 16, 16)]
                    growh = tabh_v[pl.ds((gm // 256) * 16, 16)]
                    gclv = growl[c15]
                    gsh = growh[jnp.maximum(iota16 - gclv, 0)]
                    cperm = jnp.where(iota16 < gclv, growl, gsh)
                    cntg = growl[15] + growh[15]
                    idvec = ids_v[pl.ds(j * 128 + g * 16, 16)]
                    ids_hit = idvec[cperm]

                    for r in range(16):
                        @pl.when(r < cntg)
                        def _(j=j, g=g, r=r, ids_hit=ids_hit, cperm=cperm):
                            idc = ids_hit[r].astype(jnp.int32)
                            tl = cperm[r]
                            p = ptr_sm[0]

                            @pl.when(p <= CAP)
                            def _(j=j, g=g, idc=idc, tl=tl, p=p):
                                rowl = tabl_v[pl.ds((idc % 256) * 16, 16)]
                                rowh = tabh_v[pl.ds((idc // 256) * 16, 16)]
                                clv = rowl[c15]
                                sh = rowh[jnp.maximum(iota16 - clv, 0)]
                                perm = jnp.where(iota16 < clv, rowl, sh)
                                o = (j * 64 + g * 16 + tl) * 16
                                dc_st[pl.ds(p, 16)] = \
                                    row_v[pl.ds(o, 16)][perm]
                                rx_st[pl.ds(p, 16)] = \
                                    px_v[pl.ds(o, 16)][perm]
                                ry_st[pl.ds(p, 16)] = \
                                    py_v[pl.ds(o, 16)][perm]
                                rz_st[pl.ds(p, 16)] = \
                                    pz_v[pl.ds(o, 16)][perm]
                                ptr_sm[0] = p + rowl[15] + rowh[15]

            return _j

        lax.fori_loop(0, NJ, jblock, 0)
        dc_st[pl.ds(ptr_sm[0], 16)] = pad
        pltpu.sync_copy(rx_st.at[pl.ds(0, CAP)], rx_out.at[a])
        pltpu.sync_copy(ry_st.at[pl.ds(0, CAP)], ry_out.at[a])
        pltpu.sync_copy(rz_st.at[pl.ds(0, CAP)], rz_out.at[a])
        pltpu.sync_copy(dc_st.at[pl.ds(0, CAP)], dc_out.at[a])
        return 0

    lax.fori_loop(0, ANCH_PER_W, per_anchor, 0)


def _run_sc_compact(d2, ids, px, py, pz, tabl, tabh):
    mesh = plsc.VectorSubcoreMesh(core_axis_name="c", subcore_axis_name="s")
    plane = jax.ShapeDtypeStruct((N_SAMPLES, CAP), jnp.float32)
    fn = pl.kernel(
        _sc_compact_body,
        out_type=(plane, plane, plane, plane),
        mesh=mesh,
        scratch_types=[
            pltpu.VMEM((N_POINTS,), jnp.float32),
            pltpu.VMEM((N_POINTS,), jnp.float32),
            pltpu.VMEM((N_POINTS,), jnp.float32),
            pltpu.VMEM((N_POINTS,), jnp.float32),
            pltpu.VMEM((NJ * 128,), jnp.float32),
            pltpu.VMEM((4096,), jnp.int32),
            pltpu.VMEM((4096,), jnp.int32),
            pltpu.VMEM((STAGE_W,), jnp.float32),
            pltpu.VMEM((STAGE_W,), jnp.float32),
            pltpu.VMEM((STAGE_W,), jnp.float32),
            pltpu.VMEM((STAGE_W,), jnp.float32),
            pltpu.SMEM((1,), jnp.int32),
        ],
    )
    return fn(d2, ids, px, py, pz, tabl, tabh)


def _mlp_kernel(pts_ref, d2_ref, fps_ref, w1_ref, b1_ref, w2_ref, b2_ref,
                out_ref):
    d2 = d2_ref[...]
    bm = d2.shape[0]

    dj = d2[:, :, None]
    dk = d2[:, None, :]
    kio = lax.broadcasted_iota(jnp.int32, (bm, CAP, CAP), 2)
    jio = lax.broadcasted_iota(jnp.int32, (bm, CAP, CAP), 1)
    t = (dk < dj) | ((dk == dj) & (kio <= jio))
    rank = jnp.sum(t.astype(jnp.int32), axis=2)
    valid = (rank <= MAXK) & (d2 <= jnp.float32(R2))

    rel3 = (pts_ref[...] - fps_ref[...][:, None, :]) / jnp.float32(RADIUS)
    rel = rel3.reshape(bm * CAP, 3)
    h = jnp.dot(rel, w1_ref[...], preferred_element_type=jnp.float32)
    h = jnp.maximum(h + b1_ref[...], 0.0)
    h2 = jnp.dot(h, w2_ref[...], preferred_element_type=jnp.float32)
    h2 = h2 + b2_ref[...]
    h3 = h2.reshape(bm, CAP, D_OUT)
    masked = jnp.where(valid[:, :, None], h3, jnp.float32(-1e30))
    out_ref[...] = jnp.max(masked, axis=1)


def _run_mlp(pts3, d2c, fps_points, W1, b1, W2, b2):
    bm = 64
    return pl.pallas_call(
        _mlp_kernel,
        grid=(N_SAMPLES // bm,),
        in_specs=[
            pl.BlockSpec((bm, CAP, 3), lambda i: (i, 0, 0)),
            pl.BlockSpec((bm, CAP), lambda i: (i, 0)),
            pl.BlockSpec((bm, 3), lambda i: (i, 0)),
            pl.BlockSpec((3, D_HID), lambda i: (0, 0)),
            pl.BlockSpec((1, D_HID), lambda i: (0, 0)),
            pl.BlockSpec((D_HID, D_OUT), lambda i: (0, 0)),
            pl.BlockSpec((1, D_OUT), lambda i: (0, 0)),
        ],
        out_specs=pl.BlockSpec((bm, D_OUT), lambda i: (i, 0)),
        out_shape=jax.ShapeDtypeStruct((N_SAMPLES, D_OUT), jnp.float32),
    )(pts3, d2c, fps_points, W1, b1.reshape(1, D_HID), W2,
      b2.reshape(1, D_OUT))


def kernel(points, batch, W1, b1, W2, b2):
    fx, fy, fz = _run_fps(points)
    fps_points = jnp.stack([fx, fy, fz], axis=-1)
    d2, ids = _run_d2(fps_points, points.T, _make_s12(), _make_s3())
    tabl, tabh = _make_perm_tabs()
    rx, ry, rz, d2c = _run_sc_compact(
        d2, ids, points[:, 0], points[:, 1], points[:, 2], tabl, tabh)
    pts3 = jnp.stack([rx, ry, rz], axis=-1)
    features = _run_mlp(pts3, d2c, fps_points, W1, b1, W2, b2)
    fps_batch = jnp.zeros((N_SAMPLES,), batch.dtype)
    return (fps_points, features, fps_batch)


# speedup vs baseline: 2.7010x; 2.7010x over previous
import functools

import jax
import jax.numpy as jnp
from jax import lax
from jax.experimental import pallas as pl
from jax.experimental.pallas import tpu as pltpu
from jax.experimental.pallas import tpu_sc as plsc

N_POINTS = 16384
NB_NEIGHBORS = 16
N_SAMPLES = N_POINTS // NB_NEIGHBORS
RADIUS = 0.08
R2 = RADIUS * RADIUS
MAXK = 32
CAP = 128
STAGE_W = 160
D_HID = 128
D_OUT = 256

NUM_WORKERS = 32
ANCH_PER_W = N_SAMPLES // NUM_WORKERS


def _fps_kernel(x_ref, y_ref, z_ref, fx_ref, fy_ref, fz_ref):
    x = x_ref[...]
    y = y_ref[...]
    z = z_ref[...]
    row_io = lax.broadcasted_iota(jnp.int32, (128, 128), 0)
    col_io = lax.broadcasted_iota(jnp.int32, (128, 128), 1)
    flat_io = row_io * 128 + col_io

    x0 = x[0, 0]
    y0 = y[0, 0]
    z0 = z[0, 0]
    dx = x - x0
    dy = y - y0
    dz = z - z0
    dists0 = (dx * dx + dy * dy) + dz * dz

    sio = lax.broadcasted_iota(jnp.int32, (8, 128), 0) * 128 + \
        lax.broadcasted_iota(jnp.int32, (8, 128), 1)
    sel0 = sio == 0
    fxv0 = jnp.where(sel0, x0, jnp.float32(0.0))
    fyv0 = jnp.where(sel0, y0, jnp.float32(0.0))
    fzv0 = jnp.where(sel0, z0, jnp.float32(0.0))

    def body(i, st):
        dists, fxv, fyv, fzv = st
        maxv = jnp.max(dists)
        nxt = jnp.min(jnp.where(dists == maxv, flat_io, jnp.int32(N_POINTS)))
        one_hot = flat_io == nxt
        px = jnp.sum(jnp.where(one_hot, x, jnp.float32(0.0)))
        py = jnp.sum(jnp.where(one_hot, y, jnp.float32(0.0)))
        pz = jnp.sum(jnp.where(one_hot, z, jnp.float32(0.0)))
        ddx = x - px
        ddy = y - py
        ddz = z - pz
        d = (ddx * ddx + ddy * ddy) + ddz * ddz
        dists = jnp.minimum(dists, d)
        sel = sio == i
        fxv = jnp.where(sel, px, fxv)
        fyv = jnp.where(sel, py, fyv)
        fzv = jnp.where(sel, pz, fzv)
        return (dists, fxv, fyv, fzv)

    _, fxv, fyv, fzv = lax.fori_loop(
        1, N_SAMPLES, body, (dists0, fxv0, fyv0, fzv0))
    fx_ref[...] = fxv
    fy_ref[...] = fyv
    fz_ref[...] = fzv


def _run_fps(points):
    x2 = points[:, 0].reshape(128, 128)
    y2 = points[:, 1].reshape(128, 128)
    z2 = points[:, 2].reshape(128, 128)
    out = jax.ShapeDtypeStruct((8, 128), jnp.float32)
    fx, fy, fz = pl.pallas_call(
        _fps_kernel,
        out_shape=(out, out, out),
    )(x2, y2, z2)
    return fx.reshape(N_SAMPLES), fy.reshape(N_SAMPLES), fz.reshape(N_SAMPLES)


CBLK = 1024
NJ = N_POINTS // CBLK
GRP = 256
NGRP_BLK = CBLK // GRP


def _make_s12():
    import numpy as np
    s = np.zeros((CBLK, 128), np.float32)
    p = np.arange(CBLK)
    s[p, p // 16] = 2.0 ** (p % 16)
    s[p, 64 + p // GRP] = 1.0
    return jnp.asarray(s, jnp.bfloat16)


def _d2_kernel(fps_ref, ptt_ref, s12_ref, d2_ref, ids_ref):
    f = fps_ref[...]
    ptt = ptt_ref[...]
    q2 = jnp.sum(f * f, axis=1, keepdims=True)
    p2 = jnp.sum(ptt * ptt, axis=0, keepdims=True)
    mm = jnp.dot(f, ptt, preferred_element_type=jnp.float32)
    d2 = jnp.maximum((q2 + p2) - 2.0 * mm, 0.0)
    d2_ref[...] = d2
    mask = (d2 <= jnp.float32(R2)).astype(jnp.bfloat16)
    ids_ref[...] = jnp.dot(mask, s12_ref[...],
                           preferred_element_type=jnp.float32)


def _run_d2(fps_points, ptt, s12):
    return pl.pallas_call(
        _d2_kernel,
        grid=(8, NJ),
        in_specs=[
            pl.BlockSpec((128, 3), lambda i, j: (i, 0)),
            pl.BlockSpec((3, CBLK), lambda i, j: (0, j)),
            pl.BlockSpec((CBLK, 128), lambda i, j: (0, 0)),
        ],
        out_specs=[
            pl.BlockSpec((128, CBLK), lambda i, j: (i, j)),
            pl.BlockSpec((128, 128), lambda i, j: (i, j)),
        ],
        out_shape=[
            jax.ShapeDtypeStruct((N_SAMPLES, N_POINTS), jnp.float32),
            jax.ShapeDtypeStruct((N_SAMPLES, NJ * 128), jnp.float32),
        ],
    )(fps_points, ptt, s12)


def _make_perm_tabs():
    import numpy as np
    tl = np.zeros((256, 16), np.int32)
    th = np.zeros((256, 16), np.int32)
    for m in range(256):
        bits = [b for b in range(8) if (m >> b) & 1]
        tl[m, :len(bits)] = bits
        th[m, :len(bits)] = [b + 8 for b in bits]
        th[m, 8:15] = 8
        tl[m, 15] = len(bits)
        th[m, 15] = len(bits)
    return jnp.asarray(tl.reshape(-1)), jnp.asarray(th.reshape(-1))


def _sc_compact_body(d2_hbm, ids_hbm, px_hbm, py_hbm, pz_hbm,
                     tabl_hbm, tabh_hbm,
                     rx_out, ry_out, rz_out, dc_out,
                     px_v, py_v, pz_v, row_v, ids_v, tabl_v, tabh_v,
                     rx_st, ry_st, rz_st, dc_st, ptr_sm):
    wid = lax.axis_index("c") * 16 + lax.axis_index("s")
    base = wid * ANCH_PER_W

    pltpu.sync_copy(px_hbm, px_v)
    pltpu.sync_copy(py_hbm, py_v)
    pltpu.sync_copy(pz_hbm, pz_v)
    pltpu.sync_copy(tabl_hbm, tabl_v)
    pltpu.sync_copy(tabh_hbm, tabh_v)

    pad = jnp.full((16,), 1e30, jnp.float32)
    iota16 = lax.iota(jnp.int32, 16)
    c15 = jnp.full((16,), 15, jnp.int32)

    def per_anchor(i, _):
        a = base + i
        pltpu.sync_copy(d2_hbm.at[a], row_v)
        pltpu.sync_copy(ids_hbm.at[a], ids_v)
        o0 = i * CAP
        ptr_sm[0] = o0
        for k in range(CAP // 16):
            dc_st[pl.ds(o0 + k * 16, 16)] = pad

        def jblock(j, _j):
            gvec = ids_v[pl.ds(j * 128 + 64, 16)]
            for g in range(NGRP_BLK):
                gcnt = gvec[g]

                @pl.when(gcnt > jnp.float32(0))
                def _(j=j, g=g):
                    idvec = ids_v[pl.ds(j * 128 + g * 16, 16)]
                    for t in range(16):
                        idf = idvec[t]
                        p = ptr_sm[0]

                        @pl.when((idf > jnp.float32(0)) &
                                 (p <= i * CAP + CAP))
                        def _(j=j, g=g, t=t, idf=idf, p=p):
                            idc = idf.astype(jnp.int32)
                            rowl = tabl_v[pl.ds((idc % 256) * 16, 16)]
                            rowh = tabh_v[pl.ds((idc // 256) * 16, 16)]
                            clv = rowl[c15]
                            sh = rowh[jnp.maximum(iota16 - clv, 0)]
                            perm = jnp.where(iota16 < clv, rowl, sh)
                            o = (j * 64 + g * 16 + t) * 16
                            dc_st[pl.ds(p, 16)] = row_v[pl.ds(o, 16)][perm]
                            rx_st[pl.ds(p, 16)] = px_v[pl.ds(o, 16)][perm]
                            ry_st[pl.ds(p, 16)] = py_v[pl.ds(o, 16)][perm]
                            rz_st[pl.ds(p, 16)] = pz_v[pl.ds(o, 16)][perm]
                            ptr_sm[0] = p + rowl[15] + rowh[15]

            return _j

        lax.fori_loop(0, NJ, jblock, 0)
        dc_st[pl.ds(ptr_sm[0], 16)] = pad
        return 0

    lax.fori_loop(0, ANCH_PER_W, per_anchor, 0)

    nb = ANCH_PER_W * CAP
    ob = base * CAP
    pltpu.sync_copy(rx_st.at[pl.ds(0, nb)], rx_out.at[pl.ds(ob, nb)])
    pltpu.sync_copy(ry_st.at[pl.ds(0, nb)], ry_out.at[pl.ds(ob, nb)])
    pltpu.sync_copy(rz_st.at[pl.ds(0, nb)], rz_out.at[pl.ds(ob, nb)])
    pltpu.sync_copy(dc_st.at[pl.ds(0, nb)], dc_out.at[pl.ds(ob, nb)])


_OSTG = ANCH_PER_W * CAP + 32


def _run_sc_compact(d2, ids, px, py, pz, tabl, tabh):
    mesh = plsc.VectorSubcoreMesh(core_axis_name="c", subcore_axis_name="s")
    plane = jax.ShapeDtypeStruct((N_SAMPLES * CAP,), jnp.float32)
    fn = pl.kernel(
        _sc_compact_body,
        out_type=(plane, plane, plane, plane),
        mesh=mesh,
        scratch_types=[
            pltpu.VMEM((N_POINTS,), jnp.float32),
            pltpu.VMEM((N_POINTS,), jnp.float32),
            pltpu.VMEM((N_POINTS,), jnp.float32),
            pltpu.VMEM((N_POINTS,), jnp.float32),
            pltpu.VMEM((NJ * 128,), jnp.float32),
            pltpu.VMEM((4096,), jnp.int32),
            pltpu.VMEM((4096,), jnp.int32),
            pltpu.VMEM((_OSTG,), jnp.float32),
            pltpu.VMEM((_OSTG,), jnp.float32),
            pltpu.VMEM((_OSTG,), jnp.float32),
            pltpu.VMEM((_OSTG,), jnp.float32),
            pltpu.SMEM((1,), jnp.int32),
        ],
    )
    return fn(d2, ids, px, py, pz, tabl, tabh)


def _mlp_kernel(pts_ref, d2_ref, fps_ref, w1_ref, b1_ref, w2_ref, b2_ref,
                out_ref):
    d2 = d2_ref[...]
    bm = d2.shape[0]

    dj = d2[:, :, None]
    dk = d2[:, None, :]
    kio = lax.broadcasted_iota(jnp.int32, (bm, CAP, CAP), 2)
    jio = lax.broadcasted_iota(jnp.int32, (bm, CAP, CAP), 1)
    t = (dk < dj) | ((dk == dj) & (kio <= jio))
    rank = jnp.sum(t.astype(jnp.int32), axis=2)
    valid = (rank <= MAXK) & (d2 <= jnp.float32(R2))

    rel3 = (pts_ref[...] - fps_ref[...][:, None, :]) / jnp.float32(RADIUS)
    rel = rel3.reshape(bm * CAP, 3)
    h = jnp.dot(rel, w1_ref[...], preferred_element_type=jnp.float32)
    h = jnp.maximum(h + b1_ref[...], 0.0)
    h2 = jnp.dot(h, w2_ref[...], preferred_element_type=jnp.float32)
    h2 = h2 + b2_ref[...]
    h3 = h2.reshape(bm, CAP, D_OUT)
    masked = jnp.where(valid[:, :, None], h3, jnp.float32(-1e30))
    out_ref[...] = jnp.max(masked, axis=1)


def _run_mlp(pts3, d2c, fps_points, W1, b1, W2, b2):
    bm = 64
    return pl.pallas_call(
        _mlp_kernel,
        grid=(N_SAMPLES // bm,),
        in_specs=[
            pl.BlockSpec((bm, CAP, 3), lambda i: (i, 0, 0)),
            pl.BlockSpec((bm, CAP), lambda i: (i, 0)),
            pl.BlockSpec((bm, 3), lambda i: (i, 0)),
            pl.BlockSpec((3, D_HID), lambda i: (0, 0)),
            pl.BlockSpec((1, D_HID), lambda i: (0, 0)),
            pl.BlockSpec((D_HID, D_OUT), lambda i: (0, 0)),
            pl.BlockSpec((1, D_OUT), lambda i: (0, 0)),
        ],
        out_specs=pl.BlockSpec((bm, D_OUT), lambda i: (i, 0)),
        out_shape=jax.ShapeDtypeStruct((N_SAMPLES, D_OUT), jnp.float32),
    )(pts3, d2c, fps_points, W1, b1.reshape(1, D_HID), W2,
      b2.reshape(1, D_OUT))


def kernel(points, batch, W1, b1, W2, b2):
    fx, fy, fz = _run_fps(points)
    fps_points = jnp.stack([fx, fy, fz], axis=-1)
    d2, ids = _run_d2(fps_points, points.T, _make_s12())
    tabl, tabh = _make_perm_tabs()
    rx, ry, rz, d2c = _run_sc_compact(
        d2, ids, points[:, 0], points[:, 1], points[:, 2], tabl, tabh)
    rx = rx.reshape(N_SAMPLES, CAP)
    ry = ry.reshape(N_SAMPLES, CAP)
    rz = rz.reshape(N_SAMPLES, CAP)
    d2c = d2c.reshape(N_SAMPLES, CAP)
    pts3 = jnp.stack([rx, ry, rz], axis=-1)
    features = _run_mlp(pts3, d2c, fps_points, W1, b1, W2, b2)
    fps_batch = jnp.zeros((N_SAMPLES,), batch.dtype)
    return (fps_points, features, fps_batch)


# speedup vs baseline: 3.4884x; 1.2915x over previous
import functools

import jax
import jax.numpy as jnp
from jax import lax
from jax.experimental import pallas as pl
from jax.experimental.pallas import tpu as pltpu
from jax.experimental.pallas import tpu_sc as plsc

N_POINTS = 16384
NB_NEIGHBORS = 16
N_SAMPLES = N_POINTS // NB_NEIGHBORS
RADIUS = 0.08
R2 = RADIUS * RADIUS
MAXK = 32
CAP = 128
STAGE_W = 160
D_HID = 128
D_OUT = 256

NUM_WORKERS = 32
ANCH_PER_W = N_SAMPLES // NUM_WORKERS


def _fps_kernel(x_ref, y_ref, z_ref, fx_ref, fy_ref, fz_ref):
    x = x_ref[...]
    y = y_ref[...]
    z = z_ref[...]
    row_io = lax.broadcasted_iota(jnp.int32, (128, 128), 0)
    col_io = lax.broadcasted_iota(jnp.int32, (128, 128), 1)
    flat_io = row_io * 128 + col_io

    x0 = x[0, 0]
    y0 = y[0, 0]
    z0 = z[0, 0]
    dx = x - x0
    dy = y - y0
    dz = z - z0
    dists0 = (dx * dx + dy * dy) + dz * dz

    sio = lax.broadcasted_iota(jnp.int32, (8, 128), 0) * 128 + \
        lax.broadcasted_iota(jnp.int32, (8, 128), 1)
    sel0 = sio == 0
    fxv0 = jnp.where(sel0, x0, jnp.float32(0.0))
    fyv0 = jnp.where(sel0, y0, jnp.float32(0.0))
    fzv0 = jnp.where(sel0, z0, jnp.float32(0.0))

    def body(i, st):
        dists, fxv, fyv, fzv = st
        maxv = jnp.max(dists)
        nxt = jnp.min(jnp.where(dists == maxv, flat_io, jnp.int32(N_POINTS)))
        one_hot = flat_io == nxt
        px = jnp.sum(jnp.where(one_hot, x, jnp.float32(0.0)))
        py = jnp.sum(jnp.where(one_hot, y, jnp.float32(0.0)))
        pz = jnp.sum(jnp.where(one_hot, z, jnp.float32(0.0)))
        ddx = x - px
        ddy = y - py
        ddz = z - pz
        d = (ddx * ddx + ddy * ddy) + ddz * ddz
        dists = jnp.minimum(dists, d)
        sel = sio == i
        fxv = jnp.where(sel, px, fxv)
        fyv = jnp.where(sel, py, fyv)
        fzv = jnp.where(sel, pz, fzv)
        return (dists, fxv, fyv, fzv)

    _, fxv, fyv, fzv = lax.fori_loop(
        1, N_SAMPLES, body, (dists0, fxv0, fyv0, fzv0))
    fx_ref[...] = fxv
    fy_ref[...] = fyv
    fz_ref[...] = fzv


def _run_fps(points):
    x2 = points[:, 0].reshape(128, 128)
    y2 = points[:, 1].reshape(128, 128)
    z2 = points[:, 2].reshape(128, 128)
    out = jax.ShapeDtypeStruct((8, 128), jnp.float32)
    fx, fy, fz = pl.pallas_call(
        _fps_kernel,
        out_shape=(out, out, out),
    )(x2, y2, z2)
    return fx.reshape(N_SAMPLES), fy.reshape(N_SAMPLES), fz.reshape(N_SAMPLES)


CBLK = 1024
NJ = N_POINTS // CBLK
GRP = 256
NGRP_BLK = CBLK // GRP


def _make_s12():
    import numpy as np
    s = np.zeros((CBLK, 128), np.float32)
    p = np.arange(CBLK)
    s[p, p // 16] = 2.0 ** (p % 16)
    s[p, 64 + p // GRP] = 1.0
    return jnp.asarray(s, jnp.bfloat16)


def _d2_kernel(fps_ref, ptt_ref, s12_ref, d2_ref, ids_ref):
    f = fps_ref[...]
    ptt = ptt_ref[...]
    q2 = jnp.sum(f * f, axis=1, keepdims=True)
    p2 = jnp.sum(ptt * ptt, axis=0, keepdims=True)
    mm = jnp.dot(f, ptt, preferred_element_type=jnp.float32)
    d2 = jnp.maximum((q2 + p2) - 2.0 * mm, 0.0)
    d2_ref[...] = d2
    mask = (d2 <= jnp.float32(R2)).astype(jnp.bfloat16)
    ids_ref[...] = jnp.dot(mask, s12_ref[...],
                           preferred_element_type=jnp.float32)


def _run_d2(fps_points, ptt, s12):
    return pl.pallas_call(
        _d2_kernel,
        grid=(8, NJ),
        in_specs=[
            pl.BlockSpec((128, 3), lambda i, j: (i, 0)),
            pl.BlockSpec((3, CBLK), lambda i, j: (0, j)),
            pl.BlockSpec((CBLK, 128), lambda i, j: (0, 0)),
        ],
        out_specs=[
            pl.BlockSpec((128, CBLK), lambda i, j: (i, j)),
            pl.BlockSpec((128, 128), lambda i, j: (i, j)),
        ],
        out_shape=[
            jax.ShapeDtypeStruct((N_SAMPLES, N_POINTS), jnp.float32),
            jax.ShapeDtypeStruct((N_SAMPLES, NJ * 128), jnp.float32),
        ],
    )(fps_points, ptt, s12)


def _make_perm_tabs():
    import numpy as np
    tl = np.zeros((256, 16), np.int32)
    th = np.zeros((256, 16), np.int32)
    for m in range(256):
        bits = [b for b in range(8) if (m >> b) & 1]
        tl[m, :len(bits)] = bits
        th[m, :len(bits)] = [b + 8 for b in bits]
        th[m, 8:15] = 8
        tl[m, 15] = len(bits)
        th[m, 15] = len(bits)
    return jnp.asarray(tl.reshape(-1)), jnp.asarray(th.reshape(-1))


def _sc_compact_body(d2_hbm, ids_hbm, px_hbm, py_hbm, pz_hbm,
                     tabl_hbm, tabh_hbm,
                     rx_out, ry_out, rz_out, dc_out,
                     px_v, py_v, pz_v, row_v, ids_v, tabl_v, tabh_v,
                     rx_st, ry_st, rz_st, dc_st, ptr_sm):
    wid = lax.axis_index("c") * 16 + lax.axis_index("s")
    base = wid * ANCH_PER_W

    pltpu.sync_copy(px_hbm, px_v)
    pltpu.sync_copy(py_hbm, py_v)
    pltpu.sync_copy(pz_hbm, pz_v)
    pltpu.sync_copy(tabl_hbm, tabl_v)
    pltpu.sync_copy(tabh_hbm, tabh_v)

    pad = jnp.full((16,), 1e30, jnp.float32)
    iota16 = lax.iota(jnp.int32, 16)
    c15 = jnp.full((16,), 15, jnp.int32)

    def per_anchor(i, _):
        a = base + i
        pltpu.sync_copy(d2_hbm.at[a], row_v)
        pltpu.sync_copy(ids_hbm.at[a], ids_v)
        o0 = i * CAP
        ptr_sm[0] = o0
        for k in range(CAP // 16):
            dc_st[pl.ds(o0 + k * 16, 16)] = pad

        def jblock(j, _j):
            gvec = ids_v[pl.ds(j * 128 + 64, 16)]
            for g in range(NGRP_BLK):
                gcnt = gvec[g]

                @pl.when(gcnt > jnp.float32(0))
                def _(j=j, g=g):
                    idvec = ids_v[pl.ds(j * 128 + g * 16, 16)]
                    for t in range(16):
                        idf = idvec[t]

                        @pl.when(idf > jnp.float32(0))
                        def _(j=j, g=g, t=t, idf=idf):
                            p = ptr_sm[0]

                            @pl.when(p <= i * CAP + CAP)
                            def _(j=j, g=g, t=t, idf=idf, p=p):
                                idc = idf.astype(jnp.int32)
                                rowl = tabl_v[pl.ds((idc % 256) * 16, 16)]
                                rowh = tabh_v[pl.ds((idc // 256) * 16, 16)]
                                clv = rowl[c15]
                                sh = rowh[jnp.maximum(iota16 - clv, 0)]
                                perm = jnp.where(iota16 < clv, rowl, sh)
                                o = (j * 64 + g * 16 + t) * 16
                                dc_st[pl.ds(p, 16)] = \
                                    row_v[pl.ds(o, 16)][perm]
                                rx_st[pl.ds(p, 16)] = \
                                    px_v[pl.ds(o, 16)][perm]
                                ry_st[pl.ds(p, 16)] = \
                                    py_v[pl.ds(o, 16)][perm]
                                rz_st[pl.ds(p, 16)] = \
                                    pz_v[pl.ds(o, 16)][perm]
                                ptr_sm[0] = p + rowl[15] + rowh[15]

            return _j

        lax.fori_loop(0, NJ, jblock, 0)
        dc_st[pl.ds(ptr_sm[0], 16)] = pad
        return 0

    lax.fori_loop(0, ANCH_PER_W, per_anchor, 0)

    nb = ANCH_PER_W * CAP
    ob = base * CAP
    pltpu.sync_copy(rx_st.at[pl.ds(0, nb)], rx_out.at[pl.ds(ob, nb)])
    pltpu.sync_copy(ry_st.at[pl.ds(0, nb)], ry_out.at[pl.ds(ob, nb)])
    pltpu.sync_copy(rz_st.at[pl.ds(0, nb)], rz_out.at[pl.ds(ob, nb)])
    pltpu.sync_copy(dc_st.at[pl.ds(0, nb)], dc_out.at[pl.ds(ob, nb)])


_OSTG = ANCH_PER_W * CAP + 32


def _run_sc_compact(d2, ids, px, py, pz, tabl, tabh):
    mesh = plsc.VectorSubcoreMesh(core_axis_name="c", subcore_axis_name="s")
    plane = jax.ShapeDtypeStruct((N_SAMPLES * CAP,), jnp.float32)
    fn = pl.kernel(
        _sc_compact_body,
        out_type=(plane, plane, plane, plane),
        mesh=mesh,
        scratch_types=[
            pltpu.VMEM((N_POINTS,), jnp.float32),
            pltpu.VMEM((N_POINTS,), jnp.float32),
            pltpu.VMEM((N_POINTS,), jnp.float32),
            pltpu.VMEM((N_POINTS,), jnp.float32),
            pltpu.VMEM((NJ * 128,), jnp.float32),
            pltpu.VMEM((4096,), jnp.int32),
            pltpu.VMEM((4096,), jnp.int32),
            pltpu.VMEM((_OSTG,), jnp.float32),
            pltpu.VMEM((_OSTG,), jnp.float32),
            pltpu.VMEM((_OSTG,), jnp.float32),
            pltpu.VMEM((_OSTG,), jnp.float32),
            pltpu.SMEM((1,), jnp.int32),
        ],
    )
    return fn(d2, ids, px, py, pz, tabl, tabh)


def _mlp_kernel(pts_ref, d2_ref, fps_ref, w1_ref, b1_ref, w2_ref, b2_ref,
                out_ref):
    d2 = d2_ref[...]
    bm = d2.shape[0]

    dj = d2[:, :, None]
    dk = d2[:, None, :]
    kio = lax.broadcasted_iota(jnp.int32, (bm, CAP, CAP), 2)
    jio = lax.broadcasted_iota(jnp.int32, (bm, CAP, CAP), 1)
    t = (dk < dj) | ((dk == dj) & (kio <= jio))
    rank = jnp.sum(t.astype(jnp.int32), axis=2)
    valid = (rank <= MAXK) & (d2 <= jnp.float32(R2))

    rel3 = (pts_ref[...] - fps_ref[...][:, None, :]) / jnp.float32(RADIUS)
    rel = rel3.reshape(bm * CAP, 3)
    h = jnp.dot(rel, w1_ref[...], preferred_element_type=jnp.float32)
    h = jnp.maximum(h + b1_ref[...], 0.0)
    h2 = jnp.dot(h, w2_ref[...], preferred_element_type=jnp.float32)
    h2 = h2 + b2_ref[...]
    h3 = h2.reshape(bm, CAP, D_OUT)
    masked = jnp.where(valid[:, :, None], h3, jnp.float32(-1e30))
    out_ref[...] = jnp.max(masked, axis=1)


def _run_mlp(pts3, d2c, fps_points, W1, b1, W2, b2):
    bm = 64
    return pl.pallas_call(
        _mlp_kernel,
        grid=(N_SAMPLES // bm,),
        in_specs=[
            pl.BlockSpec((bm, CAP, 3), lambda i: (i, 0, 0)),
            pl.BlockSpec((bm, CAP), lambda i: (i, 0)),
            pl.BlockSpec((bm, 3), lambda i: (i, 0)),
            pl.BlockSpec((3, D_HID), lambda i: (0, 0)),
            pl.BlockSpec((1, D_HID), lambda i: (0, 0)),
            pl.BlockSpec((D_HID, D_OUT), lambda i: (0, 0)),
            pl.BlockSpec((1, D_OUT), lambda i: (0, 0)),
        ],
        out_specs=pl.BlockSpec((bm, D_OUT), lambda i: (i, 0)),
        out_shape=jax.ShapeDtypeStruct((N_SAMPLES, D_OUT), jnp.float32),
    )(pts3, d2c, fps_points, W1, b1.reshape(1, D_HID), W2,
      b2.reshape(1, D_OUT))


def kernel(points, batch, W1, b1, W2, b2):
    fx, fy, fz = _run_fps(points)
    fps_points = jnp.stack([fx, fy, fz], axis=-1)
    d2, ids = _run_d2(fps_points, points.T, _make_s12())
    tabl, tabh = _make_perm_tabs()
    rx, ry, rz, d2c = _run_sc_compact(
        d2, ids, points[:, 0], points[:, 1], points[:, 2], tabl, tabh)
    rx = rx.reshape(N_SAMPLES, CAP)
    ry = ry.reshape(N_SAMPLES, CAP)
    rz = rz.reshape(N_SAMPLES, CAP)
    d2c = d2c.reshape(N_SAMPLES, CAP)
    pts3 = jnp.stack([rx, ry, rz], axis=-1)
    features = _run_mlp(pts3, d2c, fps_points, W1, b1, W2, b2)
    fps_batch = jnp.zeros((N_SAMPLES,), batch.dtype)
    return (fps_points, features, fps_batch)


# speedup vs baseline: 4.2378x; 1.2148x over previous
import functools

import jax
import jax.numpy as jnp
from jax import lax
from jax.experimental import pallas as pl
from jax.experimental.pallas import tpu as pltpu
from jax.experimental.pallas import tpu_sc as plsc

N_POINTS = 16384
NB_NEIGHBORS = 16
N_SAMPLES = N_POINTS // NB_NEIGHBORS
RADIUS = 0.08
R2 = RADIUS * RADIUS
MAXK = 32
CAP = 128
STAGE_W = 160
D_HID = 128
D_OUT = 256

NUM_WORKERS = 32
ANCH_PER_W = N_SAMPLES // NUM_WORKERS


def _fps_kernel(x_ref, y_ref, z_ref, fx_ref, fy_ref, fz_ref):
    x = x_ref[...]
    y = y_ref[...]
    z = z_ref[...]
    row_io = lax.broadcasted_iota(jnp.int32, (128, 128), 0)
    col_io = lax.broadcasted_iota(jnp.int32, (128, 128), 1)
    flat_io = row_io * 128 + col_io

    x0 = x[0, 0]
    y0 = y[0, 0]
    z0 = z[0, 0]
    dx = x - x0
    dy = y - y0
    dz = z - z0
    dists0 = (dx * dx + dy * dy) + dz * dz

    sio = lax.broadcasted_iota(jnp.int32, (8, 128), 0) * 128 + \
        lax.broadcasted_iota(jnp.int32, (8, 128), 1)
    sel0 = sio == 0
    fxv0 = jnp.where(sel0, x0, jnp.float32(0.0))
    fyv0 = jnp.where(sel0, y0, jnp.float32(0.0))
    fzv0 = jnp.where(sel0, z0, jnp.float32(0.0))

    def body(i, st):
        dists, fxv, fyv, fzv = st
        maxv = jnp.max(dists)
        nxt = jnp.min(jnp.where(dists == maxv, flat_io, jnp.int32(N_POINTS)))
        one_hot = flat_io == nxt
        px = jnp.sum(jnp.where(one_hot, x, jnp.float32(0.0)))
        py = jnp.sum(jnp.where(one_hot, y, jnp.float32(0.0)))
        pz = jnp.sum(jnp.where(one_hot, z, jnp.float32(0.0)))
        ddx = x - px
        ddy = y - py
        ddz = z - pz
        d = (ddx * ddx + ddy * ddy) + ddz * ddz
        dists = jnp.minimum(dists, d)
        sel = sio == i
        fxv = jnp.where(sel, px, fxv)
        fyv = jnp.where(sel, py, fyv)
        fzv = jnp.where(sel, pz, fzv)
        return (dists, fxv, fyv, fzv)

    _, fxv, fyv, fzv = lax.fori_loop(
        1, N_SAMPLES, body, (dists0, fxv0, fyv0, fzv0))
    fx_ref[...] = fxv
    fy_ref[...] = fyv
    fz_ref[...] = fzv


def _run_fps(points):
    x2 = points[:, 0].reshape(128, 128)
    y2 = points[:, 1].reshape(128, 128)
    z2 = points[:, 2].reshape(128, 128)
    out = jax.ShapeDtypeStruct((8, 128), jnp.float32)
    fx, fy, fz = pl.pallas_call(
        _fps_kernel,
        out_shape=(out, out, out),
    )(x2, y2, z2)
    return fx.reshape(N_SAMPLES), fy.reshape(N_SAMPLES), fz.reshape(N_SAMPLES)


CBLK = 1024
NJ = N_POINTS // CBLK
GRP = 256
NGRP_BLK = CBLK // GRP


def _make_s12():
    import numpy as np
    s = np.zeros((CBLK, 128), np.float32)
    p = np.arange(CBLK)
    s[p, p // 16] = 2.0 ** (p % 16)
    s[p, 64 + p // GRP] = 1.0
    s[p, 68 + p // 64] = 1.0
    return jnp.asarray(s, jnp.bfloat16)


def _d2_kernel(fps_ref, ptt_ref, s12_ref, d2_ref, ids_ref):
    f = fps_ref[...]
    ptt = ptt_ref[...]
    q2 = jnp.sum(f * f, axis=1, keepdims=True)
    p2 = jnp.sum(ptt * ptt, axis=0, keepdims=True)
    mm = jnp.dot(f, ptt, preferred_element_type=jnp.float32)
    d2 = jnp.maximum((q2 + p2) - 2.0 * mm, 0.0)
    d2_ref[...] = d2
    mask = (d2 <= jnp.float32(R2)).astype(jnp.bfloat16)
    ids_ref[...] = jnp.dot(mask, s12_ref[...],
                           preferred_element_type=jnp.float32)


def _run_d2(fps_points, ptt, s12):
    return pl.pallas_call(
        _d2_kernel,
        grid=(8, NJ),
        in_specs=[
            pl.BlockSpec((128, 3), lambda i, j: (i, 0)),
            pl.BlockSpec((3, CBLK), lambda i, j: (0, j)),
            pl.BlockSpec((CBLK, 128), lambda i, j: (0, 0)),
        ],
        out_specs=[
            pl.BlockSpec((128, CBLK), lambda i, j: (i, j)),
            pl.BlockSpec((128, 128), lambda i, j: (i, j)),
        ],
        out_shape=[
            jax.ShapeDtypeStruct((N_SAMPLES, N_POINTS), jnp.float32),
            jax.ShapeDtypeStruct((N_SAMPLES, NJ * 128), jnp.float32),
        ],
    )(fps_points, ptt, s12)


def _make_perm_tabs():
    import numpy as np
    tl = np.zeros((256, 16), np.int32)
    th = np.zeros((256, 16), np.int32)
    for m in range(256):
        bits = [b for b in range(8) if (m >> b) & 1]
        tl[m, :len(bits)] = bits
        th[m, :len(bits)] = [b + 8 for b in bits]
        th[m, 8:15] = 8
        tl[m, 15] = len(bits)
        th[m, 15] = len(bits)
    return jnp.asarray(tl.reshape(-1)), jnp.asarray(th.reshape(-1))


def _sc_compact_body(d2_hbm, ids_hbm, px_hbm, py_hbm, pz_hbm,
                     tabl_hbm, tabh_hbm,
                     rx_out, ry_out, rz_out, dc_out,
                     px_v, py_v, pz_v, row_v, ids_v, tabl_v, tabh_v,
                     rx_st, ry_st, rz_st, dc_st, ptr_sm):
    wid = lax.axis_index("c") * 16 + lax.axis_index("s")
    base = wid * ANCH_PER_W

    pltpu.sync_copy(px_hbm, px_v)
    pltpu.sync_copy(py_hbm, py_v)
    pltpu.sync_copy(pz_hbm, pz_v)
    pltpu.sync_copy(tabl_hbm, tabl_v)
    pltpu.sync_copy(tabh_hbm, tabh_v)

    pad = jnp.full((16,), 1e30, jnp.float32)
    iota16 = lax.iota(jnp.int32, 16)
    c15 = jnp.full((16,), 15, jnp.int32)

    def per_anchor(i, _):
        a = base + i
        pltpu.sync_copy(d2_hbm.at[a], row_v)
        pltpu.sync_copy(ids_hbm.at[a], ids_v)
        o0 = i * CAP
        ptr_sm[0] = o0
        for k in range(CAP // 16):
            dc_st[pl.ds(o0 + k * 16, 16)] = pad

        def jblock(j, _j):
            gvec = ids_v[pl.ds(j * 128 + 64, 16)]
            for g in range(NGRP_BLK):
                gcnt = gvec[g]

                @pl.when(gcnt > jnp.float32(0))
                def _(j=j, g=g):
                    idvec = ids_v[pl.ds(j * 128 + g * 16, 16)]
                    qv = ids_v[pl.ds(j * 128 + 68, 16)]
                    for qq in range(4):
                        qf = qv[4 * g + qq]

                        @pl.when(qf > jnp.float32(0))
                        def _(j=j, g=g, qq=qq, idvec=idvec):
                            for tt in range(4):
                                t = qq * 4 + tt
                                idf = idvec[t]

                                @pl.when(idf > jnp.float32(0))
                                def _(j=j, g=g, t=t, idf=idf):
                                    p = ptr_sm[0]

                                    @pl.when(p <= i * CAP + CAP)
                                    def _(j=j, g=g, t=t, idf=idf, p=p):
                                        idc = idf.astype(jnp.int32)
                                        rowl = tabl_v[
                                            pl.ds((idc % 256) * 16, 16)]
                                        rowh = tabh_v[
                                            pl.ds((idc // 256) * 16, 16)]
                                        clv = rowl[c15]
                                        sh = rowh[
                                            jnp.maximum(iota16 - clv, 0)]
                                        perm = jnp.where(
                                            iota16 < clv, rowl, sh)
                                        o = (j * 64 + g * 16 + t) * 16
                                        dc_st[pl.ds(p, 16)] = \
                                            row_v[pl.ds(o, 16)][perm]
                                        rx_st[pl.ds(p, 16)] = \
                                            px_v[pl.ds(o, 16)][perm]
                                        ry_st[pl.ds(p, 16)] = \
                                            py_v[pl.ds(o, 16)][perm]
                                        rz_st[pl.ds(p, 16)] = \
                                            pz_v[pl.ds(o, 16)][perm]
                                        ptr_sm[0] = \
                                            p + rowl[15] + rowh[15]

            return _j

        lax.fori_loop(0, NJ, jblock, 0)
        dc_st[pl.ds(ptr_sm[0], 16)] = pad
        return 0

    lax.fori_loop(0, ANCH_PER_W, per_anchor, 0)

    nb = ANCH_PER_W * CAP
    ob = base * CAP
    pltpu.sync_copy(rx_st.at[pl.ds(0, nb)], rx_out.at[pl.ds(ob, nb)])
    pltpu.sync_copy(ry_st.at[pl.ds(0, nb)], ry_out.at[pl.ds(ob, nb)])
    pltpu.sync_copy(rz_st.at[pl.ds(0, nb)], rz_out.at[pl.ds(ob, nb)])
    pltpu.sync_copy(dc_st.at[pl.ds(0, nb)], dc_out.at[pl.ds(ob, nb)])


_OSTG = ANCH_PER_W * CAP + 32


def _run_sc_compact(d2, ids, px, py, pz, tabl, tabh):
    mesh = plsc.VectorSubcoreMesh(core_axis_name="c", subcore_axis_name="s")
    plane = jax.ShapeDtypeStruct((N_SAMPLES * CAP,), jnp.float32)
    fn = pl.kernel(
        _sc_compact_body,
        out_type=(plane, plane, plane, plane),
        mesh=mesh,
        scratch_types=[
            pltpu.VMEM((N_POINTS,), jnp.float32),
            pltpu.VMEM((N_POINTS,), jnp.float32),
            pltpu.VMEM((N_POINTS,), jnp.float32),
            pltpu.VMEM((N_POINTS,), jnp.float32),
            pltpu.VMEM((NJ * 128,), jnp.float32),
            pltpu.VMEM((4096,), jnp.int32),
            pltpu.VMEM((4096,), jnp.int32),
            pltpu.VMEM((_OSTG,), jnp.float32),
            pltpu.VMEM((_OSTG,), jnp.float32),
            pltpu.VMEM((_OSTG,), jnp.float32),
            pltpu.VMEM((_OSTG,), jnp.float32),
            pltpu.SMEM((1,), jnp.int32),
        ],
    )
    return fn(d2, ids, px, py, pz, tabl, tabh)


def _mlp_kernel(pts_ref, d2_ref, fps_ref, w1_ref, b1_ref, w2_ref, b2_ref,
                out_ref):
    d2 = d2_ref[...]
    bm = d2.shape[0]

    dj = d2[:, :, None]
    dk = d2[:, None, :]
    kio = lax.broadcasted_iota(jnp.int32, (bm, CAP, CAP), 2)
    jio = lax.broadcasted_iota(jnp.int32, (bm, CAP, CAP), 1)
    t = (dk < dj) | ((dk == dj) & (kio <= jio))
    rank = jnp.sum(t.astype(jnp.int32), axis=2)
    valid = (rank <= MAXK) & (d2 <= jnp.float32(R2))

    rel3 = (pts_ref[...] - fps_ref[...][:, None, :]) / jnp.float32(RADIUS)
    rel = rel3.reshape(bm * CAP, 3)
    h = jnp.dot(rel, w1_ref[...], preferred_element_type=jnp.float32)
    h = jnp.maximum(h + b1_ref[...], 0.0)
    h2 = jnp.dot(h, w2_ref[...], preferred_element_type=jnp.float32)
    h2 = h2 + b2_ref[...]
    h3 = h2.reshape(bm, CAP, D_OUT)
    masked = jnp.where(valid[:, :, None], h3, jnp.float32(-1e30))
    out_ref[...] = jnp.max(masked, axis=1)


def _run_mlp(pts3, d2c, fps_points, W1, b1, W2, b2):
    bm = 64
    return pl.pallas_call(
        _mlp_kernel,
        grid=(N_SAMPLES // bm,),
        in_specs=[
            pl.BlockSpec((bm, CAP, 3), lambda i: (i, 0, 0)),
            pl.BlockSpec((bm, CAP), lambda i: (i, 0)),
            pl.BlockSpec((bm, 3), lambda i: (i, 0)),
            pl.BlockSpec((3, D_HID), lambda i: (0, 0)),
            pl.BlockSpec((1, D_HID), lambda i: (0, 0)),
            pl.BlockSpec((D_HID, D_OUT), lambda i: (0, 0)),
            pl.BlockSpec((1, D_OUT), lambda i: (0, 0)),
        ],
        out_specs=pl.BlockSpec((bm, D_OUT), lambda i: (i, 0)),
        out_shape=jax.ShapeDtypeStruct((N_SAMPLES, D_OUT), jnp.float32),
    )(pts3, d2c, fps_points, W1, b1.reshape(1, D_HID), W2,
      b2.reshape(1, D_OUT))


def kernel(points, batch, W1, b1, W2, b2):
    fx, fy, fz = _run_fps(points)
    fps_points = jnp.stack([fx, fy, fz], axis=-1)
    d2, ids = _run_d2(fps_points, points.T, _make_s12())
    tabl, tabh = _make_perm_tabs()
    rx, ry, rz, d2c = _run_sc_compact(
        d2, ids, points[:, 0], points[:, 1], points[:, 2], tabl, tabh)
    rx = rx.reshape(N_SAMPLES, CAP)
    ry = ry.reshape(N_SAMPLES, CAP)
    rz = rz.reshape(N_SAMPLES, CAP)
    d2c = d2c.reshape(N_SAMPLES, CAP)
    pts3 = jnp.stack([rx, ry, rz], axis=-1)
    features = _run_mlp(pts3, d2c, fps_points, W1, b1, W2, b2)
    fps_batch = jnp.zeros((N_SAMPLES,), batch.dtype)
    return (fps_points, features, fps_batch)


# speedup vs baseline: 4.2611x; 1.0055x over previous
import functools

import jax
import jax.numpy as jnp
from jax import lax
from jax.experimental import pallas as pl
from jax.experimental.pallas import tpu as pltpu
from jax.experimental.pallas import tpu_sc as plsc

N_POINTS = 16384
NB_NEIGHBORS = 16
N_SAMPLES = N_POINTS // NB_NEIGHBORS
RADIUS = 0.08
R2 = RADIUS * RADIUS
MAXK = 32
CAP = 128
STAGE_W = 160
D_HID = 128
D_OUT = 256

NUM_WORKERS = 32
ANCH_PER_W = N_SAMPLES // NUM_WORKERS


def _fps_kernel(x_ref, y_ref, z_ref, fx_ref, fy_ref, fz_ref):
    x = x_ref[...]
    y = y_ref[...]
    z = z_ref[...]
    row_io = lax.broadcasted_iota(jnp.int32, (128, 128), 0)
    col_io = lax.broadcasted_iota(jnp.int32, (128, 128), 1)
    flat_io = row_io * 128 + col_io

    x0 = x[0, 0]
    y0 = y[0, 0]
    z0 = z[0, 0]
    dx = x - x0
    dy = y - y0
    dz = z - z0
    dists0 = (dx * dx + dy * dy) + dz * dz

    sio = lax.broadcasted_iota(jnp.int32, (8, 128), 0) * 128 + \
        lax.broadcasted_iota(jnp.int32, (8, 128), 1)
    sel0 = sio == 0
    fxv0 = jnp.where(sel0, x0, jnp.float32(0.0))
    fyv0 = jnp.where(sel0, y0, jnp.float32(0.0))
    fzv0 = jnp.where(sel0, z0, jnp.float32(0.0))

    col1 = lax.broadcasted_iota(jnp.int32, (1, 128), 1)

    def body(i, st):
        dists, fxv, fyv, fzv = st
        maxv = jnp.max(dists)
        nxt = jnp.min(jnp.where(dists == maxv, flat_io, jnp.int32(N_POINTS)))
        r = nxt // 128
        c = nxt - r * 128
        oh = col1 == c
        px = jnp.sum(jnp.where(oh, x_ref[pl.ds(r, 1), :], jnp.float32(0.0)))
        py = jnp.sum(jnp.where(oh, y_ref[pl.ds(r, 1), :], jnp.float32(0.0)))
        pz = jnp.sum(jnp.where(oh, z_ref[pl.ds(r, 1), :], jnp.float32(0.0)))
        ddx = x - px
        ddy = y - py
        ddz = z - pz
        d = (ddx * ddx + ddy * ddy) + ddz * ddz
        dists = jnp.minimum(dists, d)
        sel = sio == i
        fxv = jnp.where(sel, px, fxv)
        fyv = jnp.where(sel, py, fyv)
        fzv = jnp.where(sel, pz, fzv)
        return (dists, fxv, fyv, fzv)

    _, fxv, fyv, fzv = lax.fori_loop(
        1, N_SAMPLES, body, (dists0, fxv0, fyv0, fzv0))
    fx_ref[...] = fxv
    fy_ref[...] = fyv
    fz_ref[...] = fzv


def _run_fps(points):
    x2 = points[:, 0].reshape(128, 128)
    y2 = points[:, 1].reshape(128, 128)
    z2 = points[:, 2].reshape(128, 128)
    out = jax.ShapeDtypeStruct((8, 128), jnp.float32)
    fx, fy, fz = pl.pallas_call(
        _fps_kernel,
        out_shape=(out, out, out),
    )(x2, y2, z2)
    return fx.reshape(N_SAMPLES), fy.reshape(N_SAMPLES), fz.reshape(N_SAMPLES)


CBLK = 1024
NJ = N_POINTS // CBLK
GRP = 256
NGRP_BLK = CBLK // GRP


def _make_s12():
    import numpy as np
    s = np.zeros((CBLK, 128), np.float32)
    p = np.arange(CBLK)
    s[p, p // 16] = 2.0 ** (p % 16)
    s[p, 64 + p // GRP] = 1.0
    s[p, 68 + p // 64] = 1.0
    return jnp.asarray(s, jnp.bfloat16)


def _d2_kernel(fps_ref, ptt_ref, s12_ref, d2_ref, ids_ref):
    f = fps_ref[...]
    ptt = ptt_ref[...]
    q2 = jnp.sum(f * f, axis=1, keepdims=True)
    p2 = jnp.sum(ptt * ptt, axis=0, keepdims=True)
    mm = jnp.dot(f, ptt, preferred_element_type=jnp.float32)
    d2 = jnp.maximum((q2 + p2) - 2.0 * mm, 0.0)
    d2_ref[...] = d2
    mask = (d2 <= jnp.float32(R2)).astype(jnp.bfloat16)
    ids_ref[...] = jnp.dot(mask, s12_ref[...],
                           preferred_element_type=jnp.float32)


def _run_d2(fps_points, ptt, s12):
    return pl.pallas_call(
        _d2_kernel,
        grid=(8, NJ),
        in_specs=[
            pl.BlockSpec((128, 3), lambda i, j: (i, 0)),
            pl.BlockSpec((3, CBLK), lambda i, j: (0, j)),
            pl.BlockSpec((CBLK, 128), lambda i, j: (0, 0)),
        ],
        out_specs=[
            pl.BlockSpec((128, CBLK), lambda i, j: (i, j)),
            pl.BlockSpec((128, 128), lambda i, j: (i, j)),
        ],
        out_shape=[
            jax.ShapeDtypeStruct((N_SAMPLES, N_POINTS), jnp.float32),
            jax.ShapeDtypeStruct((N_SAMPLES, NJ * 128), jnp.float32),
        ],
    )(fps_points, ptt, s12)


def _make_perm_tabs():
    import numpy as np
    tl = np.zeros((256, 16), np.int32)
    th = np.zeros((256, 16), np.int32)
    for m in range(256):
        bits = [b for b in range(8) if (m >> b) & 1]
        tl[m, :len(bits)] = bits
        th[m, :len(bits)] = [b + 8 for b in bits]
        th[m, 8:15] = 8
        tl[m, 15] = len(bits)
        th[m, 15] = len(bits)
    return jnp.asarray(tl.reshape(-1)), jnp.asarray(th.reshape(-1))


def _sc_compact_body(d2_hbm, ids_hbm, px_hbm, py_hbm, pz_hbm,
                     tabl_hbm, tabh_hbm,
                     rx_out, ry_out, rz_out, dc_out,
                     px_v, py_v, pz_v, row_v, ids_v, tabl_v, tabh_v,
                     rx_st, ry_st, rz_st, dc_st, ptr_sm):
    wid = lax.axis_index("c") * 16 + lax.axis_index("s")
    base = wid * ANCH_PER_W

    pltpu.sync_copy(px_hbm, px_v)
    pltpu.sync_copy(py_hbm, py_v)
    pltpu.sync_copy(pz_hbm, pz_v)
    pltpu.sync_copy(tabl_hbm, tabl_v)
    pltpu.sync_copy(tabh_hbm, tabh_v)

    pad = jnp.full((16,), 1e30, jnp.float32)
    iota16 = lax.iota(jnp.int32, 16)
    c15 = jnp.full((16,), 15, jnp.int32)

    def per_anchor(i, _):
        a = base + i
        pltpu.sync_copy(d2_hbm.at[a], row_v)
        pltpu.sync_copy(ids_hbm.at[a], ids_v)
        o0 = i * CAP
        ptr_sm[0] = o0
        for k in range(CAP // 16):
            dc_st[pl.ds(o0 + k * 16, 16)] = pad

        def jblock(j, _j):
            gvec = ids_v[pl.ds(j * 128 + 64, 16)]
            for g in range(NGRP_BLK):
                gcnt = gvec[g]

                @pl.when(gcnt > jnp.float32(0))
                def _(j=j, g=g):
                    idvec = ids_v[pl.ds(j * 128 + g * 16, 16)]
                    qv = ids_v[pl.ds(j * 128 + 68, 16)]
                    for qq in range(4):
                        qf = qv[4 * g + qq]

                        @pl.when(qf > jnp.float32(0))
                        def _(j=j, g=g, qq=qq, idvec=idvec):
                            for tt in range(4):
                                t = qq * 4 + tt
                                idf = idvec[t]

                                @pl.when(idf > jnp.float32(0))
                                def _(j=j, g=g, t=t, idf=idf):
                                    p = ptr_sm[0]

                                    @pl.when(p <= i * CAP + CAP)
                                    def _(j=j, g=g, t=t, idf=idf, p=p):
                                        idc = idf.astype(jnp.int32)
                                        rowl = tabl_v[
                                            pl.ds((idc % 256) * 16, 16)]
                                        rowh = tabh_v[
                                            pl.ds((idc // 256) * 16, 16)]
                                        clv = rowl[c15]
                                        sh = rowh[
                                            jnp.maximum(iota16 - clv, 0)]
                                        perm = jnp.where(
                                            iota16 < clv, rowl, sh)
                                        o = (j * 64 + g * 16 + t) * 16
                                        dc_st[pl.ds(p, 16)] = \
                                            row_v[pl.ds(o, 16)][perm]
                                        rx_st[pl.ds(p, 16)] = \
                                            px_v[pl.ds(o, 16)][perm]
                                        ry_st[pl.ds(p, 16)] = \
                                            py_v[pl.ds(o, 16)][perm]
                                        rz_st[pl.ds(p, 16)] = \
                                            pz_v[pl.ds(o, 16)][perm]
                                        ptr_sm[0] = \
                                            p + rowl[15] + rowh[15]

            return _j

        lax.fori_loop(0, NJ, jblock, 0)
        dc_st[pl.ds(ptr_sm[0], 16)] = pad
        return 0

    lax.fori_loop(0, ANCH_PER_W, per_anchor, 0)

    nb = ANCH_PER_W * CAP
    ob = base * CAP
    pltpu.sync_copy(rx_st.at[pl.ds(0, nb)], rx_out.at[pl.ds(ob, nb)])
    pltpu.sync_copy(ry_st.at[pl.ds(0, nb)], ry_out.at[pl.ds(ob, nb)])
    pltpu.sync_copy(rz_st.at[pl.ds(0, nb)], rz_out.at[pl.ds(ob, nb)])
    pltpu.sync_copy(dc_st.at[pl.ds(0, nb)], dc_out.at[pl.ds(ob, nb)])


_OSTG = ANCH_PER_W * CAP + 32


def _run_sc_compact(d2, ids, px, py, pz, tabl, tabh):
    mesh = plsc.VectorSubcoreMesh(core_axis_name="c", subcore_axis_name="s")
    plane = jax.ShapeDtypeStruct((N_SAMPLES * CAP,), jnp.float32)
    fn = pl.kernel(
        _sc_compact_body,
        out_type=(plane, plane, plane, plane),
        mesh=mesh,
        scratch_types=[
            pltpu.VMEM((N_POINTS,), jnp.float32),
            pltpu.VMEM((N_POINTS,), jnp.float32),
            pltpu.VMEM((N_POINTS,), jnp.float32),
            pltpu.VMEM((N_POINTS,), jnp.float32),
            pltpu.VMEM((NJ * 128,), jnp.float32),
            pltpu.VMEM((4096,), jnp.int32),
            pltpu.VMEM((4096,), jnp.int32),
            pltpu.VMEM((_OSTG,), jnp.float32),
            pltpu.VMEM((_OSTG,), jnp.float32),
            pltpu.VMEM((_OSTG,), jnp.float32),
            pltpu.VMEM((_OSTG,), jnp.float32),
            pltpu.SMEM((1,), jnp.int32),
        ],
    )
    return fn(d2, ids, px, py, pz, tabl, tabh)


def _mlp_kernel(pts_ref, d2_ref, fps_ref, w1_ref, b1_ref, w2_ref, b2_ref,
                out_ref):
    d2 = d2_ref[...]
    bm = d2.shape[0]

    dj = d2[:, :, None]
    dk = d2[:, None, :]
    kio = lax.broadcasted_iota(jnp.int32, (bm, CAP, CAP), 2)
    jio = lax.broadcasted_iota(jnp.int32, (bm, CAP, CAP), 1)
    t = (dk < dj) | ((dk == dj) & (kio <= jio))
    rank = jnp.sum(t.astype(jnp.int32), axis=2)
    valid = (rank <= MAXK) & (d2 <= jnp.float32(R2))

    rel3 = (pts_ref[...] - fps_ref[...][:, None, :]) / jnp.float32(RADIUS)
    rel = rel3.reshape(bm * CAP, 3)
    h = jnp.dot(rel, w1_ref[...], preferred_element_type=jnp.float32)
    h = jnp.maximum(h + b1_ref[...], 0.0)
    h2 = jnp.dot(h, w2_ref[...], preferred_element_type=jnp.float32)
    h2 = h2 + b2_ref[...]
    h3 = h2.reshape(bm, CAP, D_OUT)
    masked = jnp.where(valid[:, :, None], h3, jnp.float32(-1e30))
    out_ref[...] = jnp.max(masked, axis=1)


def _run_mlp(pts3, d2c, fps_points, W1, b1, W2, b2):
    bm = 64
    return pl.pallas_call(
        _mlp_kernel,
        grid=(N_SAMPLES // bm,),
        in_specs=[
            pl.BlockSpec((bm, CAP, 3), lambda i: (i, 0, 0)),
            pl.BlockSpec((bm, CAP), lambda i: (i, 0)),
            pl.BlockSpec((bm, 3), lambda i: (i, 0)),
            pl.BlockSpec((3, D_HID), lambda i: (0, 0)),
            pl.BlockSpec((1, D_HID), lambda i: (0, 0)),
            pl.BlockSpec((D_HID, D_OUT), lambda i: (0, 0)),
            pl.BlockSpec((1, D_OUT), lambda i: (0, 0)),
        ],
        out_specs=pl.BlockSpec((bm, D_OUT), lambda i: (i, 0)),
        out_shape=jax.ShapeDtypeStruct((N_SAMPLES, D_OUT), jnp.float32),
    )(pts3, d2c, fps_points, W1, b1.reshape(1, D_HID), W2,
      b2.reshape(1, D_OUT))


def kernel(points, batch, W1, b1, W2, b2):
    fx, fy, fz = _run_fps(points)
    fps_points = jnp.stack([fx, fy, fz], axis=-1)
    d2, ids = _run_d2(fps_points, points.T, _make_s12())
    tabl, tabh = _make_perm_tabs()
    rx, ry, rz, d2c = _run_sc_compact(
        d2, ids, points[:, 0], points[:, 1], points[:, 2], tabl, tabh)
    rx = rx.reshape(N_SAMPLES, CAP)
    ry = ry.reshape(N_SAMPLES, CAP)
    rz = rz.reshape(N_SAMPLES, CAP)
    d2c = d2c.reshape(N_SAMPLES, CAP)
    pts3 = jnp.stack([rx, ry, rz], axis=-1)
    features = _run_mlp(pts3, d2c, fps_points, W1, b1, W2, b2)
    fps_batch = jnp.zeros((N_SAMPLES,), batch.dtype)
    return (fps_points, features, fps_batch)


# speedup vs baseline: 4.2653x; 1.0010x over previous
import jax
import jax.numpy as jnp
from jax import lax
from jax.experimental import pallas as pl
from jax.experimental.pallas import tpu as pltpu
from jax.experimental.pallas import tpu_sc as plsc

N_POINTS = 16384
NB_NEIGHBORS = 16
N_SAMPLES = N_POINTS // NB_NEIGHBORS
RADIUS = 0.08
R2 = RADIUS * RADIUS
MAXK = 32
CAP = 128
STAGE_W = 160
D_HID = 128
D_OUT = 256

NUM_WORKERS = 32
ANCH_PER_W = N_SAMPLES // NUM_WORKERS


def _fps_kernel(x_ref, y_ref, z_ref, fx_ref, fy_ref, fz_ref):
    x = x_ref[...]
    y = y_ref[...]
    z = z_ref[...]
    row_io = lax.broadcasted_iota(jnp.int32, (128, 128), 0)
    col_io = lax.broadcasted_iota(jnp.int32, (128, 128), 1)
    flat_io = row_io * 128 + col_io

    x0 = x[0, 0]
    y0 = y[0, 0]
    z0 = z[0, 0]
    dx = x - x0
    dy = y - y0
    dz = z - z0
    dists0 = (dx * dx + dy * dy) + dz * dz

    sio = lax.broadcasted_iota(jnp.int32, (8, 128), 0) * 128 + \
        lax.broadcasted_iota(jnp.int32, (8, 128), 1)
    sel0 = sio == 0
    fxv0 = jnp.where(sel0, x0, jnp.float32(0.0))
    fyv0 = jnp.where(sel0, y0, jnp.float32(0.0))
    fzv0 = jnp.where(sel0, z0, jnp.float32(0.0))

    col1 = lax.broadcasted_iota(jnp.int32, (1, 128), 1)

    def body(i, st):
        dists, fxv, fyv, fzv = st
        maxv = jnp.max(dists)
        nxt = jnp.min(jnp.where(dists == maxv, flat_io, jnp.int32(N_POINTS)))
        r = nxt // 128
        c = nxt - r * 128
        oh = col1 == c
        px = jnp.sum(jnp.where(oh, x_ref[pl.ds(r, 1), :], jnp.float32(0.0)))
        py = jnp.sum(jnp.where(oh, y_ref[pl.ds(r, 1), :], jnp.float32(0.0)))
        pz = jnp.sum(jnp.where(oh, z_ref[pl.ds(r, 1), :], jnp.float32(0.0)))
        ddx = x - px
        ddy = y - py
        ddz = z - pz
        d = (ddx * ddx + ddy * ddy) + ddz * ddz
        dists = jnp.minimum(dists, d)
        sel = sio == i
        fxv = jnp.where(sel, px, fxv)
        fyv = jnp.where(sel, py, fyv)
        fzv = jnp.where(sel, pz, fzv)
        return (dists, fxv, fyv, fzv)

    _, fxv, fyv, fzv = lax.fori_loop(
        1, N_SAMPLES, body, (dists0, fxv0, fyv0, fzv0))
    fx_ref[...] = fxv
    fy_ref[...] = fyv
    fz_ref[...] = fzv


def _run_fps(points):
    x2 = points[:, 0].reshape(128, 128)
    y2 = points[:, 1].reshape(128, 128)
    z2 = points[:, 2].reshape(128, 128)
    out = jax.ShapeDtypeStruct((8, 128), jnp.float32)
    fx, fy, fz = pl.pallas_call(
        _fps_kernel,
        out_shape=(out, out, out),
    )(x2, y2, z2)
    return fx.reshape(N_SAMPLES), fy.reshape(N_SAMPLES), fz.reshape(N_SAMPLES)


CBLK = 1024
NJ = N_POINTS // CBLK
GRP = 256
NGRP_BLK = CBLK // GRP


def _make_s12():
    import numpy as np
    s = np.zeros((CBLK, 128), np.float32)
    p = np.arange(CBLK)
    s[p, p // 16] = 2.0 ** (p % 16)
    s[p, 64 + p // GRP] = 1.0
    s[p, 68 + p // 64] = 1.0
    return jnp.asarray(s, jnp.bfloat16)


def _d2_kernel(fps_ref, ptt_ref, s12_ref, d2_ref, ids_ref):
    f = fps_ref[...]
    ptt = ptt_ref[...]
    q2 = jnp.sum(f * f, axis=1, keepdims=True)
    p2 = jnp.sum(ptt * ptt, axis=0, keepdims=True)
    mm = jnp.dot(f, ptt, preferred_element_type=jnp.float32)
    d2 = jnp.maximum((q2 + p2) - 2.0 * mm, 0.0)
    d2_ref[...] = d2
    mask = (d2 <= jnp.float32(R2)).astype(jnp.bfloat16)
    ids_ref[...] = jnp.dot(mask, s12_ref[...],
                           preferred_element_type=jnp.float32)


def _run_d2(fps_points, ptt, s12):
    return pl.pallas_call(
        _d2_kernel,
        grid=(8, NJ),
        in_specs=[
            pl.BlockSpec((128, 3), lambda i, j: (i, 0)),
            pl.BlockSpec((3, CBLK), lambda i, j: (0, j)),
            pl.BlockSpec((CBLK, 128), lambda i, j: (0, 0)),
        ],
        out_specs=[
            pl.BlockSpec((128, CBLK), lambda i, j: (i, j)),
            pl.BlockSpec((128, 128), lambda i, j: (i, j)),
        ],
        out_shape=[
            jax.ShapeDtypeStruct((N_SAMPLES, N_POINTS), jnp.float32),
            jax.ShapeDtypeStruct((N_SAMPLES, NJ * 128), jnp.float32),
        ],
    )(fps_points, ptt, s12)


def _make_perm_tabs():
    import numpy as np
    tl = np.zeros((256, 16), np.int32)
    th = np.zeros((256, 16), np.int32)
    for m in range(256):
        bits = [b for b in range(8) if (m >> b) & 1]
        tl[m, :len(bits)] = bits
        th[m, :len(bits)] = [b + 8 for b in bits]
        th[m, 8:15] = 8
        tl[m, 15] = len(bits)
        th[m, 15] = len(bits)
    return jnp.asarray(tl.reshape(-1)), jnp.asarray(th.reshape(-1))


def _sc_compact_body(d2_hbm, ids_hbm, px_hbm, py_hbm, pz_hbm,
                     tabl_hbm, tabh_hbm,
                     rx_out, ry_out, rz_out, dc_out,
                     px_v, py_v, pz_v, row_v, ids_v, tabl_v, tabh_v,
                     rx_st, ry_st, rz_st, dc_st, ptr_sm):
    wid = lax.axis_index("c") * 16 + lax.axis_index("s")
    base = wid * ANCH_PER_W

    pltpu.sync_copy(px_hbm, px_v)
    pltpu.sync_copy(py_hbm, py_v)
    pltpu.sync_copy(pz_hbm, pz_v)
    pltpu.sync_copy(tabl_hbm, tabl_v)
    pltpu.sync_copy(tabh_hbm, tabh_v)

    pad = jnp.full((16,), 1e30, jnp.float32)
    iota16 = lax.iota(jnp.int32, 16)
    c15 = jnp.full((16,), 15, jnp.int32)

    def per_anchor(i, _):
        a = base + i
        pltpu.sync_copy(d2_hbm.at[a], row_v)
        pltpu.sync_copy(ids_hbm.at[a], ids_v)
        o0 = i * CAP
        ptr_sm[0] = o0
        for k in range(CAP // 16):
            dc_st[pl.ds(o0 + k * 16, 16)] = pad

        def jblock(j, _j):
            gvec = ids_v[pl.ds(j * 128 + 64, 16)]
            for g in range(NGRP_BLK):
                gcnt = gvec[g]

                @pl.when(gcnt > jnp.float32(0))
                def _(j=j, g=g):
                    idvec = ids_v[pl.ds(j * 128 + g * 16, 16)]
                    qv = ids_v[pl.ds(j * 128 + 68, 16)]
                    for qq in range(4):
                        qf = qv[4 * g + qq]

                        @pl.when(qf > jnp.float32(0))
                        def _(j=j, g=g, qq=qq, idvec=idvec):
                            for tt in range(4):
                                t = qq * 4 + tt
                                idf = idvec[t]

                                @pl.when(idf > jnp.float32(0))
                                def _(j=j, g=g, t=t, idf=idf):
                                    p = ptr_sm[0]

                                    @pl.when(p <= i * CAP + CAP)
                                    def _(j=j, g=g, t=t, idf=idf, p=p):
                                        idc = idf.astype(jnp.int32)
                                        rowl = tabl_v[
                                            pl.ds((idc % 256) * 16, 16)]
                                        rowh = tabh_v[
                                            pl.ds((idc // 256) * 16, 16)]
                                        clv = rowl[c15]
                                        sh = rowh[
                                            jnp.maximum(iota16 - clv, 0)]
                                        perm = jnp.where(
                                            iota16 < clv, rowl, sh)
                                        o = (j * 64 + g * 16 + t) * 16
                                        dc_st[pl.ds(p, 16)] = \
                                            row_v[pl.ds(o, 16)][perm]
                                        rx_st[pl.ds(p, 16)] = \
                                            px_v[pl.ds(o, 16)][perm]
                                        ry_st[pl.ds(p, 16)] = \
                                            py_v[pl.ds(o, 16)][perm]
                                        rz_st[pl.ds(p, 16)] = \
                                            pz_v[pl.ds(o, 16)][perm]
                                        ptr_sm[0] = \
                                            p + rowl[15] + rowh[15]

            return _j

        lax.fori_loop(0, NJ, jblock, 0)
        dc_st[pl.ds(ptr_sm[0], 16)] = pad
        return 0

    lax.fori_loop(0, ANCH_PER_W, per_anchor, 0)

    nb = ANCH_PER_W * CAP
    ob = base * CAP
    pltpu.sync_copy(rx_st.at[pl.ds(0, nb)], rx_out.at[pl.ds(ob, nb)])
    pltpu.sync_copy(ry_st.at[pl.ds(0, nb)], ry_out.at[pl.ds(ob, nb)])
    pltpu.sync_copy(rz_st.at[pl.ds(0, nb)], rz_out.at[pl.ds(ob, nb)])
    pltpu.sync_copy(dc_st.at[pl.ds(0, nb)], dc_out.at[pl.ds(ob, nb)])


_OSTG = ANCH_PER_W * CAP + 32


def _run_sc_compact(d2, ids, px, py, pz, tabl, tabh):
    mesh = plsc.VectorSubcoreMesh(core_axis_name="c", subcore_axis_name="s")
    plane = jax.ShapeDtypeStruct((N_SAMPLES * CAP,), jnp.float32)
    fn = pl.kernel(
        _sc_compact_body,
        out_type=(plane, plane, plane, plane),
        mesh=mesh,
        scratch_types=[
            pltpu.VMEM((N_POINTS,), jnp.float32),
            pltpu.VMEM((N_POINTS,), jnp.float32),
            pltpu.VMEM((N_POINTS,), jnp.float32),
            pltpu.VMEM((N_POINTS,), jnp.float32),
            pltpu.VMEM((NJ * 128,), jnp.float32),
            pltpu.VMEM((4096,), jnp.int32),
            pltpu.VMEM((4096,), jnp.int32),
            pltpu.VMEM((_OSTG,), jnp.float32),
            pltpu.VMEM((_OSTG,), jnp.float32),
            pltpu.VMEM((_OSTG,), jnp.float32),
            pltpu.VMEM((_OSTG,), jnp.float32),
            pltpu.SMEM((1,), jnp.int32),
        ],
    )
    return fn(d2, ids, px, py, pz, tabl, tabh)


def _mlp_kernel(pts_ref, d2_ref, fps_ref, w1_ref, b1_ref, w2_ref, b2_ref,
                out_ref):
    d2 = d2_ref[...]
    bm = d2.shape[0]

    dj = d2[:, :, None]
    dk = d2[:, None, :]
    kio = lax.broadcasted_iota(jnp.int32, (bm, CAP, CAP), 2)
    jio = lax.broadcasted_iota(jnp.int32, (bm, CAP, CAP), 1)
    t = (dk < dj) | ((dk == dj) & (kio <= jio))
    rank = jnp.sum(t.astype(jnp.int32), axis=2)
    valid = (rank <= MAXK) & (d2 <= jnp.float32(R2))

    rel3 = (pts_ref[...] - fps_ref[...][:, None, :]) / jnp.float32(RADIUS)
    rel = rel3.reshape(bm * CAP, 3)
    h = jnp.dot(rel, w1_ref[...], preferred_element_type=jnp.float32)
    h = jnp.maximum(h + b1_ref[...], 0.0)
    h2 = jnp.dot(h, w2_ref[...], preferred_element_type=jnp.float32)
    h2 = h2 + b2_ref[...]
    h3 = h2.reshape(bm, CAP, D_OUT)
    masked = jnp.where(valid[:, :, None], h3, jnp.float32(-1e30))
    out_ref[...] = jnp.max(masked, axis=1)


def _run_mlp(pts3, d2c, fps_points, W1, b1, W2, b2):
    bm = 64
    return pl.pallas_call(
        _mlp_kernel,
        grid=(N_SAMPLES // bm,),
        in_specs=[
            pl.BlockSpec((bm, CAP, 3), lambda i: (i, 0, 0)),
            pl.BlockSpec((bm, CAP), lambda i: (i, 0)),
            pl.BlockSpec((bm, 3), lambda i: (i, 0)),
            pl.BlockSpec((3, D_HID), lambda i: (0, 0)),
            pl.BlockSpec((1, D_HID), lambda i: (0, 0)),
            pl.BlockSpec((D_HID, D_OUT), lambda i: (0, 0)),
            pl.BlockSpec((1, D_OUT), lambda i: (0, 0)),
        ],
        out_specs=pl.BlockSpec((bm, D_OUT), lambda i: (i, 0)),
        out_shape=jax.ShapeDtypeStruct((N_SAMPLES, D_OUT), jnp.float32),
    )(pts3, d2c, fps_points, W1, b1.reshape(1, D_HID), W2,
      b2.reshape(1, D_OUT))


def kernel(points, batch, W1, b1, W2, b2):
    fx, fy, fz = _run_fps(points)
    fps_points = jnp.stack([fx, fy, fz], axis=-1)
    d2, ids = _run_d2(fps_points, points.T, _make_s12())
    tabl, tabh = _make_perm_tabs()
    rx, ry, rz, d2c = _run_sc_compact(
        d2, ids, points[:, 0], points[:, 1], points[:, 2], tabl, tabh)
    rx = rx.reshape(N_SAMPLES, CAP)
    ry = ry.reshape(N_SAMPLES, CAP)
    rz = rz.reshape(N_SAMPLES, CAP)
    d2c = d2c.reshape(N_SAMPLES, CAP)
    pts3 = jnp.stack([rx, ry, rz], axis=-1)
    features = _run_mlp(pts3, d2c, fps_points, W1, b1, W2, b2)
    fps_batch = jnp.zeros((N_SAMPLES,), batch.dtype)
    return (fps_points, features, fps_batch)
